# baseline (speedup 1.0000x reference)
import jax
import jax.numpy as jnp
from jax import lax
from jax.experimental import pallas as pl
from jax.experimental.pallas import tpu as pltpu
from jax.experimental.pallas import tpu_sc as plsc

_BATCH = 16384
_D = 64
_LANES = 16
_NC = 2
_NS = 16
_NW = _NC * _NS
_BPW = _BATCH // _NW
_PHASES = 2
_RPP = _BPW // _PHASES
_Y_HIGH = 5.5


def _body(x_hbm, uf_hbm, ub_hbm, mf_hbm, mb_hbm, out_hbm,
          xv, ufrows, mfrows, brow, res, sem):
    wid = lax.axis_index("s") * _NC + lax.axis_index("c")
    base = wid * _BPW
    pltpu.sync_copy(x_hbm.at[pl.ds(base * 2, _BPW * 2)],
                    xv.at[pl.ds(0, _BPW * 2)])

    lane = lax.iota(jnp.int32, _LANES)

    def wait_group(_):
        for _k in range(4):
            pltpu.make_async_copy(uf_hbm.at[0], ufrows.at[0], sem).wait()
            pltpu.make_async_copy(mf_hbm.at[0], mfrows.at[0], sem).wait()
            pltpu.make_async_copy(ub_hbm.at[0], brow.at[0, pl.ds(0, 1)],
                                  sem).wait()
            pltpu.make_async_copy(mb_hbm.at[0], brow.at[0, pl.ds(8, 1)],
                                  sem).wait()

    for p in range(_PHASES):
        def fire(gp, carry):
            v = xv[pl.ds(p * _RPP * 2 + gp * 8, _LANES)]
            for k in range(4):
                r = gp * 4 + k
                ru = v[2 * k]
                rm = v[2 * k + 1]
                pltpu.async_copy(uf_hbm.at[ru], ufrows.at[r], sem)
                pltpu.async_copy(mf_hbm.at[rm], mfrows.at[r], sem)
                pltpu.async_copy(ub_hbm.at[ru], brow.at[r, pl.ds(0, 1)], sem)
                pltpu.async_copy(mb_hbm.at[rm], brow.at[r, pl.ds(8, 1)], sem)

            @pl.when(gp > 0)
            def _():
                wait_group(None)

            return carry

        lax.fori_loop(0, _RPP // 4, fire, 0)
        wait_group(None)

        def group_body(g, carry):
            acc = jnp.zeros((_LANES,), jnp.float32)
            for rr in range(_LANES):
                r = g * _LANES + rr
                s = ufrows[r, pl.ds(0, _LANES)] * mfrows[r, pl.ds(0, _LANES)]
                for k in range(1, _D // _LANES):
                    s = s + (ufrows[r, pl.ds(k * _LANES, _LANES)] *
                             mfrows[r, pl.ds(k * _LANES, _LANES)])
                bv = brow[r, pl.ds(0, _LANES)]
                acc = jnp.where(lane == rr, jnp.sum(s) + bv[0] + bv[8], acc)
            res[pl.ds(p * _RPP + g * _LANES, _LANES)] = (
                _Y_HIGH / (1.0 + jnp.exp(-acc)))
            return carry

        lax.fori_loop(0, _RPP // _LANES, group_body, 0)

    pltpu.sync_copy(res, out_hbm.at[pl.ds(base, _BPW)])


@jax.jit
def kernel(x, user_factors, user_bias, movie_factors, movie_bias):
    f = pl.kernel(
        _body,
        out_type=jax.ShapeDtypeStruct((_BATCH,), jnp.float32),
        mesh=plsc.VectorSubcoreMesh(core_axis_name="c", subcore_axis_name="s"),
        compiler_params=pltpu.CompilerParams(needs_layout_passes=False),
        scratch_types=[
            pltpu.VMEM((_BPW * 2 + _LANES,), jnp.int32),
            pltpu.VMEM((_RPP, _D), jnp.float32),
            pltpu.VMEM((_RPP, _D), jnp.float32),
            pltpu.VMEM((_RPP, _LANES), jnp.float32),
            pltpu.VMEM((_BPW,), jnp.float32),
            pltpu.SemaphoreType.DMA,
        ],
    )
    out = f(x.reshape(_BATCH * 2), user_factors, user_bias,
            movie_factors, movie_bias)
    return out.reshape(_BATCH, 1)
